# Initial kernel scaffold; baseline (speedup 1.0000x reference)
import jax
import jax.numpy as jnp
from jax.experimental import pallas as pl


def kernel(logits, target_codon_ids, aa_ids, species_ids, mask, weight_matrix, ref_distributions):
    raise NotImplementedError("write your pallas kernel here")


# speedup vs baseline: 4.8269x; 4.8269x over previous
import jax
import jax.numpy as jnp
import numpy as np
from jax.experimental import pallas as pl
from jax.experimental.pallas import tpu as pltpu

_AA_STR = "FFLLSSSSYY**CC*WLLLLPPPPHHQQRRRRIIIMTTTTNNKKSSRRVVVVAAAADDEEGGGG"
_letters = sorted(set(_AA_STR))
_aa2id = {a: i + 3 for i, a in enumerate(_letters)}
_N_AA = 3 + len(_letters)
_N_CODONS = 65
_nsyn_np = np.zeros(_N_AA, dtype=np.float32)
for _a in _letters:
    _nsyn_np[_aa2id[_a]] = float(_AA_STR.count(_a))
_W_CE, _W_CAI, _W_RSCU = 1.0, 0.4, 0.3


def _loss_body(x_ref, tgt_ref, aa_ref, m_ref, soh_ref, wm_ref, rd_ref,
               nsyn_ref, num_ref, den_ref, cai_ref, kl_ref):
    b = pl.program_id(0)
    x = x_ref[0]
    tgt = tgt_ref[0]
    aa = aa_ref[0]
    m = m_ref[0]
    L, V = x.shape

    iota_v = jax.lax.broadcasted_iota(jnp.int32, (L, V), 1)

    mx = jnp.max(x, axis=1, keepdims=True)
    pidx = jnp.min(jnp.where(x == mx, iota_v, V), axis=1, keepdims=True)
    t_oh = (iota_v == tgt).astype(jnp.float32)
    p_oh = (iota_v == pidx).astype(jnp.float32)
    se = jnp.sum(jnp.exp(x - mx), axis=1, keepdims=True)
    lse = jnp.log(se) + mx
    x_t = jnp.sum(x * t_oh, axis=1, keepdims=True)
    cew = (tgt != 0).astype(jnp.float32)
    ce_num = jnp.sum((lse - x_t) * cew, keepdims=True)
    ce_den = jnp.sum(cew, keepdims=True)

    soh = soh_ref[0]
    wrow = jnp.sum(wm_ref[...] * soh, axis=0, keepdims=True)
    refrow = jnp.sum(rd_ref[...] * soh, axis=0, keepdims=True)

    cnt = jnp.maximum(jnp.sum(m, keepdims=True), 1.0)
    w_t = jnp.maximum(jnp.sum(t_oh * wrow, axis=1, keepdims=True), 1e-8)
    w_p = jnp.maximum(jnp.sum(p_oh * wrow, axis=1, keepdims=True), 1e-8)
    tcai = jnp.exp(jnp.sum(jnp.log(w_t) * m, keepdims=True) / cnt)
    pcai = jnp.exp(jnp.sum(jnp.log(w_p) * m, keepdims=True) / cnt)
    cai_term = jnp.maximum(tcai - pcai, 0.0)

    mb = m > 0.5
    aa_iota = jax.lax.broadcasted_iota(jnp.int32, (L, _N_AA), 1)
    aa_oh = (aa_iota == aa).astype(jnp.float32)
    grp_gate = (aa > 2)
    nsyn = nsyn_ref[...]

    def rscu(oh, ids):
        valid_f = ((ids > 0) & mb).astype(jnp.float32)
        counts = jnp.sum(oh * valid_f, axis=0, keepdims=True)
        ag = aa_oh * jnp.where(grp_gate, valid_f, 0.0)
        pair = jax.lax.dot_general(
            ag, oh, (((0,), (0,)), ((), ())),
            preferred_element_type=jnp.float32)
        present = (pair > 0).astype(jnp.float32)
        cu = present * counts
        totals = jnp.sum(cu, axis=1, keepdims=True)
        rp = jnp.where(totals > 0,
                       cu * nsyn / jnp.maximum(totals, 1e-12), 0.0)
        return jnp.max(rp, axis=0, keepdims=True)

    rscu_t = rscu(t_oh, tgt)
    rscu_p = rscu(p_oh, pidx)

    combined = 0.7 * rscu_t + 0.3 * refrow
    pv = rscu_p + 1e-8
    tv = combined + 1e-8
    pd = pv / jnp.sum(pv, keepdims=True)
    td = tv / jnp.sum(tv, keepdims=True)
    kl = jnp.sum(td * jnp.log(td / pd), keepdims=True)

    zero = jnp.zeros((1, 1), jnp.float32)

    @pl.when(b == 0)
    def _init():
        num_ref[...] = zero
        den_ref[...] = zero
        cai_ref[...] = zero
        kl_ref[...] = zero

    num_ref[...] += ce_num
    den_ref[...] += ce_den
    cai_ref[...] += cai_term
    kl_ref[...] += kl


def kernel(logits, target_codon_ids, aa_ids, species_ids, mask,
           weight_matrix, ref_distributions):
    B, L, V = logits.shape
    S = weight_matrix.shape[0]
    tgt3 = target_codon_ids.astype(jnp.int32).reshape(B, L, 1)
    aa3 = aa_ids.astype(jnp.int32).reshape(B, L, 1)
    m3 = mask.astype(jnp.float32).reshape(B, L, 1)
    soh3 = jax.nn.one_hot(species_ids, S, dtype=jnp.float32).reshape(B, S, 1)
    nsyn = jnp.asarray(_nsyn_np).reshape(_N_AA, 1)

    scalar = jax.ShapeDtypeStruct((1, 1), jnp.float32)
    acc_spec = pl.BlockSpec((1, 1), lambda b: (0, 0))
    num, den, cai_s, kl_s = pl.pallas_call(
        _loss_body,
        grid=(B,),
        in_specs=[
            pl.BlockSpec((1, L, V), lambda b: (b, 0, 0)),
            pl.BlockSpec((1, L, 1), lambda b: (b, 0, 0)),
            pl.BlockSpec((1, L, 1), lambda b: (b, 0, 0)),
            pl.BlockSpec((1, L, 1), lambda b: (b, 0, 0)),
            pl.BlockSpec((1, S, 1), lambda b: (b, 0, 0)),
            pl.BlockSpec((S, V), lambda b: (0, 0)),
            pl.BlockSpec((S, V), lambda b: (0, 0)),
            pl.BlockSpec((_N_AA, 1), lambda b: (0, 0)),
        ],
        out_specs=[acc_spec, acc_spec, acc_spec, acc_spec],
        out_shape=[scalar, scalar, scalar, scalar],
        compiler_params=pltpu.CompilerParams(
            dimension_semantics=("arbitrary",)),
    )(logits, tgt3, aa3, m3, soh3, weight_matrix, ref_distributions, nsyn)

    ce = num[0, 0] / jnp.maximum(den[0, 0], 1.0)
    cai_loss = cai_s[0, 0] / B
    rscu_loss = kl_s[0, 0] / B
    total = _W_CE * ce + _W_CAI * cai_loss + _W_RSCU * rscu_loss
    return (total, ce, cai_loss, rscu_loss)
